# baseline (device time: 14657 ns/iter reference)
import jax
import jax.numpy as jnp
from jax import lax
from jax.experimental import pallas as pl
from jax.experimental.pallas import tpu as pltpu

N_DEV = 16
BLK = 64


def kernel(x, w_mat):
    k_dim, my_cols = x.shape
    kw, n_dim = w_mat.shape

    def body(x_ref, w_hbm, out_ref, xt_ref, send_buf, w_vmem,
             send_sems, recv_sems, w_sem):
        my = lax.axis_index("i")

        wcp = pltpu.make_async_copy(w_hbm, w_vmem, w_sem)
        wcp.start()

        bar = pltpu.get_barrier_semaphore()
        for j in range(N_DEV):
            @pl.when(j != my)
            def _(j=j):
                pl.semaphore_signal(
                    bar, inc=1,
                    device_id=(j,), device_id_type=pl.DeviceIdType.MESH,
                )

        for j in range(N_DEV):
            send_buf[j] = x_ref[pl.ds(j * BLK, BLK), :].astype(jnp.bfloat16).T

        pl.semaphore_wait(bar, N_DEV - 1)

        for j in range(N_DEV):
            @pl.when(j != my)
            def _(j=j):
                rdma = pltpu.make_async_remote_copy(
                    src_ref=send_buf.at[j],
                    dst_ref=xt_ref.at[pl.ds(my * BLK, BLK), :],
                    send_sem=send_sems.at[j],
                    recv_sem=recv_sems.at[my],
                    device_id=(j,),
                    device_id_type=pl.DeviceIdType.MESH,
                )
                rdma.start()

        xt_ref[pl.ds(my * BLK, BLK), :] = send_buf[my]

        wcp.wait()

        for j in range(N_DEV):
            @pl.when(j != my)
            def _(j=j):
                recv = pltpu.make_async_remote_copy(
                    src_ref=send_buf.at[j],
                    dst_ref=xt_ref.at[pl.ds(j * BLK, BLK), :],
                    send_sem=send_sems.at[j],
                    recv_sem=recv_sems.at[j],
                    device_id=(my,),
                    device_id_type=pl.DeviceIdType.MESH,
                )
                recv.wait_recv()

        y = lax.dot_general(
            xt_ref[:, :].astype(jnp.float32),
            w_vmem[:, :],
            (((0,), (0,)), ((), ())),
            preferred_element_type=jnp.float32,
        )
        out_ref[:, :] = y * jax.nn.sigmoid(y)

        for j in range(N_DEV):
            @pl.when(j != my)
            def _(j=j):
                send = pltpu.make_async_remote_copy(
                    src_ref=send_buf.at[j],
                    dst_ref=xt_ref.at[pl.ds(j * BLK, BLK), :],
                    send_sem=send_sems.at[j],
                    recv_sem=recv_sems.at[j],
                    device_id=(j,),
                    device_id_type=pl.DeviceIdType.MESH,
                )
                send.wait_send()

    return pl.pallas_call(
        body,
        out_shape=jax.ShapeDtypeStruct((BLK, n_dim), jnp.float32),
        in_specs=[
            pl.BlockSpec(memory_space=pltpu.VMEM),
            pl.BlockSpec(memory_space=pltpu.MemorySpace.HBM),
        ],
        out_specs=pl.BlockSpec(memory_space=pltpu.VMEM),
        scratch_shapes=[
            pltpu.VMEM((k_dim, BLK), jnp.bfloat16),
            pltpu.VMEM((N_DEV, BLK, BLK), jnp.bfloat16),
            pltpu.VMEM((kw, n_dim), jnp.float32),
            pltpu.SemaphoreType.DMA((N_DEV,)),
            pltpu.SemaphoreType.DMA((N_DEV,)),
            pltpu.SemaphoreType.DMA,
        ],
        compiler_params=pltpu.CompilerParams(collective_id=0),
    )(x, w_mat)


# device time: 12790 ns/iter; 1.1460x vs baseline; 1.1460x over previous
import jax
import jax.numpy as jnp
from jax import lax
from jax.experimental import pallas as pl
from jax.experimental.pallas import tpu as pltpu

N_DEV = 16
BLK = 64


def kernel(x, w_mat):
    k_dim, my_cols = x.shape
    kw, n_dim = w_mat.shape

    def body(x_ref, w_hbm, out_ref, recv_buf, send_buf, w_vmem,
             send_sems, recv_sems, credit_sems, w_sem):
        my = lax.axis_index("i")

        wcp = pltpu.make_async_copy(w_hbm, w_vmem, w_sem)
        wcp.start()

        bar = pltpu.get_barrier_semaphore()
        pl.semaphore_signal(bar, inc=1)
        pl.semaphore_wait(bar, 1)

        for j in range(N_DEV):
            @pl.when(j != my)
            def _(j=j):
                pl.semaphore_signal(
                    credit_sems.at[my], inc=1,
                    device_id=(j,), device_id_type=pl.DeviceIdType.MESH,
                )

        for j in range(N_DEV):
            @pl.when(j != my)
            def _(j=j):
                send_buf[j] = x_ref[pl.ds(j * BLK, BLK), :].astype(jnp.bfloat16)
                pl.semaphore_wait(credit_sems.at[j], 1)
                rdma = pltpu.make_async_remote_copy(
                    src_ref=send_buf.at[j],
                    dst_ref=recv_buf.at[my],
                    send_sem=send_sems.at[j],
                    recv_sem=recv_sems.at[my],
                    device_id=(j,),
                    device_id_type=pl.DeviceIdType.MESH,
                )
                rdma.start()

        recv_buf[my] = x_ref[pl.ds(my * BLK, BLK), :].astype(jnp.bfloat16)

        wcp.wait()

        y = jnp.zeros((BLK, n_dim), dtype=jnp.float32)
        for j in range(N_DEV):
            @pl.when(j != my)
            def _(j=j):
                recv = pltpu.make_async_remote_copy(
                    src_ref=recv_buf.at[j],
                    dst_ref=recv_buf.at[j],
                    send_sem=send_sems.at[j],
                    recv_sem=recv_sems.at[j],
                    device_id=(my,),
                    device_id_type=pl.DeviceIdType.MESH,
                )
                recv.wait_recv()
            y = y + jnp.dot(
                recv_buf[j].astype(jnp.float32),
                w_vmem[pl.ds(j * BLK, BLK), :],
                preferred_element_type=jnp.float32,
            )
        out_ref[:, :] = y * jax.nn.sigmoid(y)

        for j in range(N_DEV):
            @pl.when(j != my)
            def _(j=j):
                send = pltpu.make_async_remote_copy(
                    src_ref=send_buf.at[j],
                    dst_ref=recv_buf.at[j],
                    send_sem=send_sems.at[j],
                    recv_sem=recv_sems.at[j],
                    device_id=(j,),
                    device_id_type=pl.DeviceIdType.MESH,
                )
                send.wait_send()

    return pl.pallas_call(
        body,
        out_shape=jax.ShapeDtypeStruct((BLK, n_dim), jnp.float32),
        in_specs=[
            pl.BlockSpec(memory_space=pltpu.VMEM),
            pl.BlockSpec(memory_space=pltpu.MemorySpace.HBM),
        ],
        out_specs=pl.BlockSpec(memory_space=pltpu.VMEM),
        scratch_shapes=[
            pltpu.VMEM((N_DEV, BLK, BLK), jnp.bfloat16),
            pltpu.VMEM((N_DEV, BLK, BLK), jnp.bfloat16),
            pltpu.VMEM((kw, n_dim), jnp.float32),
            pltpu.SemaphoreType.DMA((N_DEV,)),
            pltpu.SemaphoreType.DMA((N_DEV,)),
            pltpu.SemaphoreType.REGULAR((N_DEV,)),
            pltpu.SemaphoreType.DMA,
        ],
        compiler_params=pltpu.CompilerParams(collective_id=0),
    )(x, w_mat)
